# baseline (speedup 1.0000x reference)
import functools

import jax
import jax.numpy as jnp
from jax import lax
from jax.experimental import pallas as pl
from jax.experimental.pallas import tpu as pltpu
from jax.experimental.pallas import tpu_sc as plsc

N = 10000
E = 320000
D_IN = 128
D_OUT = 128
D_HID = 256

NC = 2
NS = 16
L = 16
CK = 96

NCH_F = -(-(E // NS) // CK)
NCH_E = -(-(E // (NC * NS)) // CK)

STRIPE = 632

_MESH = plsc.VectorSubcoreMesh(
    core_axis_name="c", subcore_axis_name="s", num_cores=NC, num_subcores=NS
)


@functools.partial(
    pl.kernel,
    mesh=_MESH,
    out_type=jax.ShapeDtypeStruct((2 * N, 128), jnp.float32),
    scratch_types=[
        pltpu.VMEM_SHARED((N + 8, 128), jnp.float32),
        pltpu.VMEM((CK,), jnp.int32),
        pltpu.VMEM((CK, 128), jnp.float32),
    ],
)
def _sc_degree(col_hbm, z_hbm, ones_hbm, out, hist, cidx, ones_v):
    c = lax.axis_index("c")
    s = lax.axis_index("s")
    r0 = jnp.minimum(s * STRIPE, N - STRIPE)
    pltpu.sync_copy(ones_hbm, ones_v)
    pltpu.sync_copy(z_hbm.at[pl.ds(r0, STRIPE)], hist.at[pl.ds(r0, STRIPE)])
    plsc.subcore_barrier()

    base0 = (c * NS + s) * NCH_E

    def body(i, carry):
        pltpu.sync_copy(col_hbm.at[pl.ds((base0 + i) * CK, CK)], cidx)
        pltpu.sync_copy(ones_v, hist.at[cidx], add=True)
        return carry

    lax.fori_loop(0, NCH_E, body, 0)
    plsc.subcore_barrier()
    pltpu.sync_copy(hist.at[pl.ds(r0, STRIPE)], out.at[pl.ds(c * N + r0, STRIPE)])


def _make_sc_scatter(feature_split):
    nchunk = NCH_F if feature_split else NCH_E

    @functools.partial(
        pl.kernel,
        mesh=_MESH,
        out_type=jax.ShapeDtypeStruct((2 * N, 128), jnp.float32),
        scratch_types=[
            pltpu.VMEM_SHARED((N + 8, 128), jnp.float32),
            pltpu.VMEM((CK,), jnp.int32),
            pltpu.VMEM((CK,), jnp.int32),
            pltpu.VMEM((CK, 128), jnp.float32),
        ],
    )
    def scat_kernel(g_hbm, row_hbm, col_hbm, z_hbm, out, acc, ridx, cidx, gbuf):
        c = lax.axis_index("c")
        s = lax.axis_index("s")
        r0 = jnp.minimum(s * STRIPE, N - STRIPE)
        pltpu.sync_copy(z_hbm.at[pl.ds(r0, STRIPE)], acc.at[pl.ds(r0, STRIPE)])
        plsc.subcore_barrier()

        if feature_split:
            rbase = (c * NS + s) * nchunk
            cbase = s * nchunk
        else:
            rbase = cbase = (c * NS + s) * nchunk

        def body(i, carry):
            pltpu.sync_copy(row_hbm.at[pl.ds((rbase + i) * CK, CK)], ridx)
            pltpu.sync_copy(col_hbm.at[pl.ds((cbase + i) * CK, CK)], cidx)
            pltpu.sync_copy(g_hbm.at[ridx], gbuf)
            pltpu.sync_copy(gbuf, acc.at[cidx], add=True)
            return carry

        lax.fori_loop(0, nchunk, body, 0)
        plsc.subcore_barrier()
        pltpu.sync_copy(acc.at[pl.ds(r0, STRIPE)], out.at[pl.ds(c * N + r0, STRIPE)])

    return scat_kernel


_sc_scatter_feat = _make_sc_scatter(True)
_sc_scatter_edge = _make_sc_scatter(False)


BN = 2000
NB = N // BN


def _dinv_block(da, db):
    cnt = (jnp.sum(da, axis=1, keepdims=True)
           + jnp.sum(db, axis=1, keepdims=True)) * (1.0 / 128.0) + 1.0
    return lax.rsqrt(cnt)


def _tc_m1(x, W1, deg2):
    def kern(x_ref, w_ref, da_ref, db_ref, g_ref):
        dinv = _dinv_block(da_ref[...], db_ref[...])
        xs = x_ref[...] * dinv
        g_ref[...] = jnp.dot(xs, w_ref[...], preferred_element_type=jnp.float32)

    return pl.pallas_call(
        kern,
        grid=(2, NB),
        in_specs=[
            pl.BlockSpec((BN, D_IN), lambda j, i: (i, 0)),
            pl.BlockSpec((D_IN, 128), lambda j, i: (0, j)),
            pl.BlockSpec((BN, 128), lambda j, i: (i, 0)),
            pl.BlockSpec((BN, 128), lambda j, i: (NB + i, 0)),
        ],
        out_specs=pl.BlockSpec((BN, 128), lambda j, i: (j * NB + i, 0)),
        out_shape=jax.ShapeDtypeStruct((2 * N, 128), jnp.float32),
    )(x, W1, deg2, deg2)


def _tc_m2(s1, g1, deg2, b1a, b1b, W2):
    def kern(sa_ref, sb_ref, ga_ref, gb_ref, da_ref, db_ref,
             ba_ref, bb_ref, w_ref, g2_ref):
        dinv = _dinv_block(da_ref[...], db_ref[...])
        z1a = jnp.maximum(dinv * (sa_ref[...] + ga_ref[...]) + ba_ref[...], 0.0)
        z1b = jnp.maximum(dinv * (sb_ref[...] + gb_ref[...]) + bb_ref[...], 0.0)
        h2 = (jnp.dot(z1a, w_ref[:128, :], preferred_element_type=jnp.float32)
              + jnp.dot(z1b, w_ref[128:, :], preferred_element_type=jnp.float32))
        g2_ref[...] = dinv * h2

    return pl.pallas_call(
        kern,
        grid=(NB,),
        in_specs=[
            pl.BlockSpec((BN, 128), lambda i: (i, 0)),
            pl.BlockSpec((BN, 128), lambda i: (NB + i, 0)),
            pl.BlockSpec((BN, 128), lambda i: (i, 0)),
            pl.BlockSpec((BN, 128), lambda i: (NB + i, 0)),
            pl.BlockSpec((BN, 128), lambda i: (i, 0)),
            pl.BlockSpec((BN, 128), lambda i: (NB + i, 0)),
            pl.BlockSpec((1, 128), lambda i: (0, 0)),
            pl.BlockSpec((1, 128), lambda i: (0, 0)),
            pl.BlockSpec((D_HID, D_OUT), lambda i: (0, 0)),
        ],
        out_specs=pl.BlockSpec((BN, D_OUT), lambda i: (i, 0)),
        out_shape=jax.ShapeDtypeStruct((N, D_OUT), jnp.float32),
    )(s1, s1, g1, g1, deg2, deg2, b1a, b1b, W2)


def _tc_m3(s2, g2, deg2, b2):
    def kern(sa_ref, sb_ref, g_ref, da_ref, db_ref, b_ref, out_ref):
        dinv = _dinv_block(da_ref[...], db_ref[...])
        out_ref[...] = jnp.maximum(
            dinv * (sa_ref[...] + sb_ref[...] + g_ref[...]) + b_ref[...], 0.0
        )

    return pl.pallas_call(
        kern,
        grid=(NB,),
        in_specs=[
            pl.BlockSpec((BN, 128), lambda i: (i, 0)),
            pl.BlockSpec((BN, 128), lambda i: (NB + i, 0)),
            pl.BlockSpec((BN, 128), lambda i: (i, 0)),
            pl.BlockSpec((BN, 128), lambda i: (i, 0)),
            pl.BlockSpec((BN, 128), lambda i: (NB + i, 0)),
            pl.BlockSpec((1, 128), lambda i: (0, 0)),
        ],
        out_specs=pl.BlockSpec((BN, D_OUT), lambda i: (i, 0)),
        out_shape=jax.ShapeDtypeStruct((N, D_OUT), jnp.float32),
    )(s2, s2, g2, deg2, deg2, b2)


def _pad_chunks(a, nspan, pad_val):
    span = E // nspan
    span_p = -(-span // CK) * CK
    a2 = jnp.pad(a.reshape(nspan, span), ((0, 0), (0, span_p - span)),
                 constant_values=pad_val)
    return a2.reshape(-1)


def kernel(x, edge_index, idx, W1, b1, W2, b2):
    del idx
    row = edge_index[0]
    col = edge_index[1]
    zeros128 = jnp.zeros((N, 128), jnp.float32)
    ones128 = jnp.ones((CK, 128), jnp.float32)
    b1a = b1[:128].reshape(1, 128)
    b1b = b1[128:].reshape(1, 128)
    b2r = b2.reshape(1, 128)

    row_f0 = _pad_chunks(row, NS, 0)
    row_f = jnp.concatenate([row_f0, row_f0 + N], axis=0)
    col_f = _pad_chunks(col, NS, N)
    row_e = _pad_chunks(row, NC * NS, 0)
    col_e = _pad_chunks(col, NC * NS, N)

    deg2 = _sc_degree(col_e, zeros128, ones128)
    g1 = _tc_m1(x, W1, deg2)
    s1 = _sc_scatter_feat(g1, row_f, col_f, zeros128)
    g2 = _tc_m2(s1, g1, deg2, b1a, b1b, W2)
    s2 = _sc_scatter_edge(g2, row_e, col_e, zeros128)
    return _tc_m3(s2, g2, deg2, b2r)


# speedup vs baseline: 1.1456x; 1.1456x over previous
import functools

import jax
import jax.numpy as jnp
from jax import lax
from jax.experimental import pallas as pl
from jax.experimental.pallas import tpu as pltpu
from jax.experimental.pallas import tpu_sc as plsc

N = 10000
E = 320000
D_IN = 128
D_OUT = 128
D_HID = 256

NC = 2
NS = 16
L = 16
CK = 96

NCH_F = (-(-(E // NS) // CK) + 1) // 2 * 2
NCH_E = (-(-(E // (NC * NS)) // CK) + 1) // 2 * 2

STRIPE = 632

_MESH = plsc.VectorSubcoreMesh(
    core_axis_name="c", subcore_axis_name="s", num_cores=NC, num_subcores=NS
)


@functools.partial(
    pl.kernel,
    mesh=_MESH,
    out_type=jax.ShapeDtypeStruct((2 * N, 128), jnp.float32),
    scratch_types=[
        pltpu.VMEM_SHARED((N + 8, 128), jnp.float32),
        pltpu.VMEM((CK,), jnp.int32),
        pltpu.VMEM((CK, 128), jnp.float32),
    ],
)
def _sc_degree(col_hbm, z_hbm, ones_hbm, out, hist, cidx, ones_v):
    c = lax.axis_index("c")
    s = lax.axis_index("s")
    r0 = jnp.minimum(s * STRIPE, N - STRIPE)
    pltpu.sync_copy(ones_hbm, ones_v)
    pltpu.sync_copy(z_hbm.at[pl.ds(r0, STRIPE)], hist.at[pl.ds(r0, STRIPE)])
    plsc.subcore_barrier()

    base0 = (c * NS + s) * NCH_E

    def body(i, carry):
        pltpu.sync_copy(col_hbm.at[pl.ds((base0 + i) * CK, CK)], cidx)
        pltpu.sync_copy(ones_v, hist.at[cidx], add=True)
        return carry

    lax.fori_loop(0, NCH_E, body, 0)
    plsc.subcore_barrier()
    pltpu.sync_copy(hist.at[pl.ds(r0, STRIPE)], out.at[pl.ds(c * N + r0, STRIPE)])


def _make_sc_scatter(feature_split):
    nchunk = NCH_F if feature_split else NCH_E

    @functools.partial(
        pl.kernel,
        mesh=_MESH,
        out_type=jax.ShapeDtypeStruct((2 * N, 128), jnp.float32),
        scratch_types=[
            pltpu.VMEM_SHARED((N + 8, 128), jnp.float32),
            pltpu.VMEM((CK,), jnp.int32),
            pltpu.VMEM((CK,), jnp.int32),
            pltpu.VMEM((CK,), jnp.int32),
            pltpu.VMEM((CK,), jnp.int32),
            pltpu.VMEM((CK, 128), jnp.float32),
            pltpu.VMEM((CK, 128), jnp.float32),
        ],
    )
    def scat_kernel(g_hbm, row_hbm, col_hbm, z_hbm, out,
                    acc, ridx0, cidx0, ridx1, cidx1, gb0, gb1):
        c = lax.axis_index("c")
        s = lax.axis_index("s")
        r0 = jnp.minimum(s * STRIPE, N - STRIPE)
        pltpu.sync_copy(z_hbm.at[pl.ds(r0, STRIPE)], acc.at[pl.ds(r0, STRIPE)])
        plsc.subcore_barrier()

        if feature_split:
            rbase = (c * NS + s) * nchunk
            cbase = s * nchunk
        else:
            rbase = cbase = (c * NS + s) * nchunk

        def load_idx(j, ridx, cidx):
            pltpu.sync_copy(row_hbm.at[pl.ds((rbase + j) * CK, CK)], ridx)
            pltpu.sync_copy(col_hbm.at[pl.ds((cbase + j) * CK, CK)], cidx)

        def run(sem0, sem1):
            load_idx(0, ridx0, cidx0)
            load_idx(1, ridx1, cidx1)
            pltpu.async_copy(g_hbm.at[ridx0], gb0, sem0)
            pltpu.async_copy(g_hbm.at[ridx1], gb1, sem1)

            def body(g, carry):
                j0 = 2 * g
                pltpu.make_async_copy(g_hbm.at[pl.ds(0, CK)], gb0, sem0).wait()
                pltpu.sync_copy(gb0, acc.at[cidx0], add=True)
                load_idx(j0 + 2, ridx0, cidx0)
                pltpu.async_copy(g_hbm.at[ridx0], gb0, sem0)
                pltpu.make_async_copy(g_hbm.at[pl.ds(0, CK)], gb1, sem1).wait()
                pltpu.sync_copy(gb1, acc.at[cidx1], add=True)
                load_idx(j0 + 3, ridx1, cidx1)
                pltpu.async_copy(g_hbm.at[ridx1], gb1, sem1)
                return carry

            lax.fori_loop(0, nchunk // 2 - 1, body, 0)
            pltpu.make_async_copy(g_hbm.at[pl.ds(0, CK)], gb0, sem0).wait()
            pltpu.sync_copy(gb0, acc.at[cidx0], add=True)
            pltpu.make_async_copy(g_hbm.at[pl.ds(0, CK)], gb1, sem1).wait()
            pltpu.sync_copy(gb1, acc.at[cidx1], add=True)

        pl.run_scoped(run, pltpu.SemaphoreType.DMA, pltpu.SemaphoreType.DMA)
        plsc.subcore_barrier()
        pltpu.sync_copy(acc.at[pl.ds(r0, STRIPE)], out.at[pl.ds(c * N + r0, STRIPE)])

    return scat_kernel


_sc_scatter_feat = _make_sc_scatter(True)
_sc_scatter_edge = _make_sc_scatter(False)


BN = 2000
NB = N // BN


def _dinv_block(da, db):
    cnt = (jnp.sum(da, axis=1, keepdims=True)
           + jnp.sum(db, axis=1, keepdims=True)) * (1.0 / 128.0) + 1.0
    return lax.rsqrt(cnt)


def _tc_m1(x, W1, deg2):
    def kern(x_ref, w_ref, da_ref, db_ref, g_ref):
        dinv = _dinv_block(da_ref[...], db_ref[...])
        xs = x_ref[...] * dinv
        g_ref[...] = jnp.dot(xs, w_ref[...], preferred_element_type=jnp.float32)

    return pl.pallas_call(
        kern,
        grid=(2, NB),
        in_specs=[
            pl.BlockSpec((BN, D_IN), lambda j, i: (i, 0)),
            pl.BlockSpec((D_IN, 128), lambda j, i: (0, j)),
            pl.BlockSpec((BN, 128), lambda j, i: (i, 0)),
            pl.BlockSpec((BN, 128), lambda j, i: (NB + i, 0)),
        ],
        out_specs=pl.BlockSpec((BN, 128), lambda j, i: (j * NB + i, 0)),
        out_shape=jax.ShapeDtypeStruct((2 * N, 128), jnp.float32),
    )(x, W1, deg2, deg2)


def _tc_m2(s1, g1, deg2, b1a, b1b, W2):
    def kern(sa_ref, sb_ref, ga_ref, gb_ref, da_ref, db_ref,
             ba_ref, bb_ref, w_ref, g2_ref):
        dinv = _dinv_block(da_ref[...], db_ref[...])
        z1a = jnp.maximum(dinv * (sa_ref[...] + ga_ref[...]) + ba_ref[...], 0.0)
        z1b = jnp.maximum(dinv * (sb_ref[...] + gb_ref[...]) + bb_ref[...], 0.0)
        h2 = (jnp.dot(z1a, w_ref[:128, :], preferred_element_type=jnp.float32)
              + jnp.dot(z1b, w_ref[128:, :], preferred_element_type=jnp.float32))
        g2_ref[...] = dinv * h2

    return pl.pallas_call(
        kern,
        grid=(NB,),
        in_specs=[
            pl.BlockSpec((BN, 128), lambda i: (i, 0)),
            pl.BlockSpec((BN, 128), lambda i: (NB + i, 0)),
            pl.BlockSpec((BN, 128), lambda i: (i, 0)),
            pl.BlockSpec((BN, 128), lambda i: (NB + i, 0)),
            pl.BlockSpec((BN, 128), lambda i: (i, 0)),
            pl.BlockSpec((BN, 128), lambda i: (NB + i, 0)),
            pl.BlockSpec((1, 128), lambda i: (0, 0)),
            pl.BlockSpec((1, 128), lambda i: (0, 0)),
            pl.BlockSpec((D_HID, D_OUT), lambda i: (0, 0)),
        ],
        out_specs=pl.BlockSpec((BN, D_OUT), lambda i: (i, 0)),
        out_shape=jax.ShapeDtypeStruct((N, D_OUT), jnp.float32),
    )(s1, s1, g1, g1, deg2, deg2, b1a, b1b, W2)


def _tc_m3(s2, g2, deg2, b2):
    def kern(sa_ref, sb_ref, g_ref, da_ref, db_ref, b_ref, out_ref):
        dinv = _dinv_block(da_ref[...], db_ref[...])
        out_ref[...] = jnp.maximum(
            dinv * (sa_ref[...] + sb_ref[...] + g_ref[...]) + b_ref[...], 0.0
        )

    return pl.pallas_call(
        kern,
        grid=(NB,),
        in_specs=[
            pl.BlockSpec((BN, 128), lambda i: (i, 0)),
            pl.BlockSpec((BN, 128), lambda i: (NB + i, 0)),
            pl.BlockSpec((BN, 128), lambda i: (i, 0)),
            pl.BlockSpec((BN, 128), lambda i: (i, 0)),
            pl.BlockSpec((BN, 128), lambda i: (NB + i, 0)),
            pl.BlockSpec((1, 128), lambda i: (0, 0)),
        ],
        out_specs=pl.BlockSpec((BN, D_OUT), lambda i: (i, 0)),
        out_shape=jax.ShapeDtypeStruct((N, D_OUT), jnp.float32),
    )(s2, s2, g2, deg2, deg2, b2)


def _pad_chunks(a, nspan, nch, pad_val):
    span = E // nspan
    a2 = jnp.pad(a.reshape(nspan, span), ((0, 0), (0, nch * CK - span)),
                 constant_values=pad_val)
    return a2.reshape(-1)


def kernel(x, edge_index, idx, W1, b1, W2, b2):
    del idx
    row = edge_index[0]
    col = edge_index[1]
    zeros128 = jnp.zeros((N, 128), jnp.float32)
    ones128 = jnp.ones((CK, 128), jnp.float32)
    b1a = b1[:128].reshape(1, 128)
    b1b = b1[128:].reshape(1, 128)
    b2r = b2.reshape(1, 128)

    row_f0 = _pad_chunks(row, NS, NCH_F, 0)
    row_f = jnp.concatenate([row_f0, row_f0 + N], axis=0)
    col_f = _pad_chunks(col, NS, NCH_F, N)
    row_e = _pad_chunks(row, NC * NS, NCH_E, 0)
    col_e = _pad_chunks(col, NC * NS, NCH_E, N)

    deg2 = _sc_degree(col_e, zeros128, ones128)
    g1 = _tc_m1(x, W1, deg2)
    s1 = _sc_scatter_feat(g1, row_f, col_f, zeros128)
    g2 = _tc_m2(s1, g1, deg2, b1a, b1b, W2)
    s2 = _sc_scatter_edge(g2, row_e, col_e, zeros128)
    return _tc_m3(s2, g2, deg2, b2r)
